# Initial kernel scaffold; baseline (speedup 1.0000x reference)
import jax
import jax.numpy as jnp
from jax.experimental import pallas as pl


def kernel(x, mask, weight, bias):
    raise NotImplementedError("write your pallas kernel here")


# speedup vs baseline: 3.2961x; 3.2961x over previous
import jax
import jax.numpy as jnp
from jax.experimental import pallas as pl

_K = 3
_CIN = 96
_COUT = 96
_H = 224
_W = 224
_R = 32
_WP = 232
_HP = 256
_HPM = 256


def _conv_body(xt_ref, xb_ref, m_ref, w_ref, b_ref, o_ref):
    g = pl.program_id(1)

    xblk = jnp.concatenate([xt_ref[0], xb_ref[0, :2]], axis=0)

    acc = jnp.zeros((_R * _W, _COUT), dtype=jnp.float32)
    for di in range(_K):
        for dj in range(_K):
            xs = xblk[di:di + _R, dj:dj + _W, :].reshape(_R * _W, _CIN)
            acc += jnp.dot(xs, w_ref[di * _K + dj],
                           preferred_element_type=jnp.float32)

    mrows = m_ref[0, pl.ds(_R * g, 40), :]
    mwin = jnp.zeros((_R, _W), dtype=jnp.float32)
    for di in range(_K):
        for dj in range(_K):
            mwin = jnp.maximum(mwin, jnp.abs(mrows[di:di + _R, dj:dj + _W]))

    out = acc.reshape(_R, _W, _COUT) + b_ref[0][None, None, :]
    out = jnp.where(mwin[:, :, None] != 0, out, 0.0)
    o_ref[0] = out


def kernel(x, mask, weight, bias):
    b = x.shape[0]
    x_nhwc = jnp.transpose(x, (0, 2, 3, 1))
    xp = jnp.pad(x_nhwc, ((0, 0), (1, _HP - _H - 1), (1, _WP - _W - 1), (0, 0)))
    mp = jnp.pad(mask[:, 0], ((0, 0), (1, _HPM - _H - 1), (1, _WP - _W - 1)))
    wk = jnp.transpose(weight, (2, 3, 1, 0)).reshape(_K * _K, _CIN, _COUT)
    b2 = bias.reshape(1, _COUT)

    grid = (b, _H // _R)
    out_nhwc = pl.pallas_call(
        _conv_body,
        grid=grid,
        in_specs=[
            pl.BlockSpec((1, _R, _WP, _CIN), lambda bb, g: (bb, g, 0, 0)),
            pl.BlockSpec((1, _R, _WP, _CIN), lambda bb, g: (bb, g + 1, 0, 0)),
            pl.BlockSpec((1, _HPM, _WP), lambda bb, g: (bb, 0, 0)),
            pl.BlockSpec((_K * _K, _CIN, _COUT), lambda bb, g: (0, 0, 0)),
            pl.BlockSpec((1, _COUT), lambda bb, g: (0, 0)),
        ],
        out_specs=pl.BlockSpec((1, _R, _W, _COUT), lambda bb, g: (bb, g, 0, 0)),
        out_shape=jax.ShapeDtypeStruct((b, _H, _W, _COUT), jnp.float32),
    )(xp, xp, mp, wk, b2)
    return jnp.transpose(out_nhwc, (0, 3, 1, 2))


# speedup vs baseline: 5.5270x; 1.6768x over previous
import jax
import jax.numpy as jnp
from jax.experimental import pallas as pl

_K = 3
_CIN = 96
_COUT = 96
_H = 224
_W = 224
_R = 32
_LB = _R * _W
_HALO = 1024
_HPAD = 256
_LPAD = _HPAD * _W


def _conv_body(xm_ref, xh_ref, mm_ref, mh_ref, w_ref, b_ref, o_ref):
    xm = xm_ref[0].astype(jnp.bfloat16)
    xh = xh_ref[0].astype(jnp.bfloat16)
    mm = mm_ref[0]
    mh = mh_ref[0]

    col = jax.lax.broadcasted_iota(jnp.int32, (1, _LB), 1) % _W
    j_first = col == 0
    j_last = col == _W - 1

    taps = []
    mwin = jnp.zeros((1, _LB), dtype=jnp.float32)
    for di in range(_K):
        for dj in range(_K):
            o = (di + 1) * _W + dj - 1
            xs = jnp.concatenate([xm[:, o:], xh[:, :o]], axis=1)
            ms = jnp.concatenate([mm[:, o:], mh[:, :o]], axis=1)
            if dj == 0:
                xs = jnp.where(j_first, jnp.bfloat16(0), xs)
                ms = jnp.where(j_first, 0.0, ms)
            elif dj == 2:
                xs = jnp.where(j_last, jnp.bfloat16(0), xs)
                ms = jnp.where(j_last, 0.0, ms)
            taps.append(xs)
            mwin = jnp.maximum(mwin, jnp.abs(ms))

    xu = jnp.concatenate(taps, axis=0)
    acc = jnp.dot(w_ref[...], xu, preferred_element_type=jnp.float32)
    out = acc + b_ref[...]
    o_ref[0] = jnp.where(mwin != 0, out, 0.0)


def kernel(x, mask, weight, bias):
    b = x.shape[0]
    xp = jnp.pad(x, ((0, 0), (0, 0), (2, _HPAD - _H - 2), (0, 0)))
    xp = xp.reshape(b, _CIN, _LPAD)
    mp = jnp.pad(mask[:, 0], ((0, 0), (2, _HPAD - _H - 2), (0, 0)))
    mp = mp.reshape(b, 1, _LPAD)
    w2 = jnp.transpose(weight, (0, 2, 3, 1)).reshape(_COUT, _K * _K * _CIN)
    w2 = w2.astype(jnp.bfloat16)
    b2 = bias.reshape(_COUT, 1)

    grid = (b, _H // _R)
    out_flat = pl.pallas_call(
        _conv_body,
        grid=grid,
        in_specs=[
            pl.BlockSpec((1, _CIN, _LB), lambda bb, g: (bb, 0, g)),
            pl.BlockSpec((1, _CIN, _HALO),
                         lambda bb, g: (bb, 0, (g + 1) * (_LB // _HALO))),
            pl.BlockSpec((1, 1, _LB), lambda bb, g: (bb, 0, g)),
            pl.BlockSpec((1, 1, _HALO),
                         lambda bb, g: (bb, 0, (g + 1) * (_LB // _HALO))),
            pl.BlockSpec((_COUT, _K * _K * _CIN), lambda bb, g: (0, 0)),
            pl.BlockSpec((_COUT, 1), lambda bb, g: (0, 0)),
        ],
        out_specs=pl.BlockSpec((1, _COUT, _LB), lambda bb, g: (bb, 0, g)),
        out_shape=jax.ShapeDtypeStruct((b, _COUT, _H * _W), jnp.float32),
    )(xp, xp, mp, mp, w2, b2)
    return out_flat.reshape(b, _COUT, _H, _W)


# speedup vs baseline: 5.5374x; 1.0019x over previous
import jax
import jax.numpy as jnp
from jax.experimental import pallas as pl
from jax.experimental.pallas import tpu as pltpu

_K = 3
_CIN = 96
_COUT = 96
_H = 224
_W = 224
_R = 32
_LB = _R * _W
_HALO = 1024
_HPAD = 256
_LPAD = _HPAD * _W


def _conv_body(xm_ref, xh_ref, mm_ref, mh_ref, w_ref, b_ref, o_ref, xu_ref):
    xm = xm_ref[0].astype(jnp.bfloat16)
    xh = xh_ref[0].astype(jnp.bfloat16)
    mm = mm_ref[0]
    mh = mh_ref[0]

    col = jax.lax.broadcasted_iota(jnp.int32, (1, _LB), 1) % _W
    j_first = col == 0
    j_last = col == _W - 1

    mwin = jnp.zeros((1, _LB), dtype=jnp.float32)
    for di in range(_K):
        for dj in range(_K):
            t = di * _K + dj
            o = (di + 1) * _W + dj - 1
            xs = jnp.concatenate([xm[:, o:], xh[:, :o]], axis=1)
            ms = jnp.concatenate([mm[:, o:], mh[:, :o]], axis=1)
            if dj == 0:
                xs = jnp.where(j_first, jnp.bfloat16(0), xs)
                ms = jnp.where(j_first, 0.0, ms)
            elif dj == 2:
                xs = jnp.where(j_last, jnp.bfloat16(0), xs)
                ms = jnp.where(j_last, 0.0, ms)
            xu_ref[t * _CIN:(t + 1) * _CIN, :] = xs
            mwin = jnp.maximum(mwin, jnp.abs(ms))

    acc = jnp.dot(w_ref[...], xu_ref[...], preferred_element_type=jnp.float32)
    out = acc + b_ref[...]
    o_ref[0] = jnp.where(mwin != 0, out, 0.0)


def kernel(x, mask, weight, bias):
    b = x.shape[0]
    xp = jnp.pad(x, ((0, 0), (0, 0), (2, _HPAD - _H - 2), (0, 0)))
    xp = xp.reshape(b, _CIN, _LPAD)
    mp = jnp.pad(mask[:, 0], ((0, 0), (2, _HPAD - _H - 2), (0, 0)))
    mp = mp.reshape(b, 1, _LPAD)
    w2 = jnp.transpose(weight, (0, 2, 3, 1)).reshape(_COUT, _K * _K * _CIN)
    w2 = w2.astype(jnp.bfloat16)
    b2 = bias.reshape(_COUT, 1)

    grid = (b, _H // _R)
    out_flat = pl.pallas_call(
        _conv_body,
        grid=grid,
        in_specs=[
            pl.BlockSpec((1, _CIN, _LB), lambda bb, g: (bb, 0, g)),
            pl.BlockSpec((1, _CIN, _HALO),
                         lambda bb, g: (bb, 0, (g + 1) * (_LB // _HALO))),
            pl.BlockSpec((1, 1, _LB), lambda bb, g: (bb, 0, g)),
            pl.BlockSpec((1, 1, _HALO),
                         lambda bb, g: (bb, 0, (g + 1) * (_LB // _HALO))),
            pl.BlockSpec((_COUT, _K * _K * _CIN), lambda bb, g: (0, 0)),
            pl.BlockSpec((_COUT, 1), lambda bb, g: (0, 0)),
        ],
        out_specs=pl.BlockSpec((1, _COUT, _LB), lambda bb, g: (bb, 0, g)),
        out_shape=jax.ShapeDtypeStruct((b, _COUT, _H * _W), jnp.float32),
        scratch_shapes=[pltpu.VMEM((_K * _K * _CIN, _LB), jnp.bfloat16)],
    )(xp, xp, mp, mp, w2, b2)
    return out_flat.reshape(b, _COUT, _H, _W)


# speedup vs baseline: 6.0056x; 1.0845x over previous
import jax
import jax.numpy as jnp
from jax.experimental import pallas as pl
from jax.experimental.pallas import tpu as pltpu

_K = 3
_CIN = 96
_COUT = 96
_H = 224
_W = 224
_R = 32
_LB = _R * _W
_HALO = 256
_L = _H * _W
_NG = _H // _R


def _conv_body(xp_ref, xm_ref, xn_ref, mp_ref, mm_ref, mn_ref, w_ref, b_ref,
               o_ref, xu_ref):
    g = pl.program_id(1)
    xall = jnp.concatenate(
        [xp_ref[0].astype(jnp.bfloat16), xm_ref[0].astype(jnp.bfloat16),
         xn_ref[0].astype(jnp.bfloat16)], axis=1)
    mall = jnp.concatenate([mp_ref[0], mm_ref[0], mn_ref[0]], axis=1)

    lane = jax.lax.broadcasted_iota(jnp.int32, (1, _LB), 1)
    col = lane % _W
    j_first = col == 0
    j_last = col == _W - 1
    row_top = jnp.logical_and(g == 0, lane < _W)
    row_bot = jnp.logical_and(g == _NG - 1, lane >= _LB - _W)

    mwin = jnp.zeros((1, _LB), dtype=jnp.float32)
    for di in range(_K):
        for dj in range(_K):
            t = di * _K + dj
            o = _HALO + (di - 1) * _W + dj - 1
            zm = jnp.zeros((1, _LB), dtype=jnp.bool_)
            if dj == 0:
                zm = j_first
            elif dj == 2:
                zm = j_last
            if di == 0:
                zm = jnp.logical_or(zm, row_top)
            elif di == 2:
                zm = jnp.logical_or(zm, row_bot)
            xs = jnp.where(zm, jnp.bfloat16(0), xall[:, o:o + _LB])
            ms = jnp.where(zm, 0.0, mall[:, o:o + _LB])
            xu_ref[t * _CIN:(t + 1) * _CIN, :] = xs
            mwin = jnp.maximum(mwin, jnp.abs(ms))

    acc = jnp.dot(w_ref[...], xu_ref[...], preferred_element_type=jnp.float32)
    out = acc + b_ref[...]
    o_ref[0] = jnp.where(mwin != 0, out, 0.0)


def kernel(x, mask, weight, bias):
    b = x.shape[0]
    xf = x.reshape(b, _CIN, _L)
    mf = mask.reshape(b, 1, _L)
    w2 = jnp.transpose(weight, (0, 2, 3, 1)).reshape(_COUT, _K * _K * _CIN)
    w2 = w2.astype(jnp.bfloat16)
    b2 = bias.reshape(_COUT, 1)

    nh = _LB // _HALO
    last_halo = _L // _HALO - 1

    grid = (b, _NG)
    out_flat = pl.pallas_call(
        _conv_body,
        grid=grid,
        in_specs=[
            pl.BlockSpec((1, _CIN, _HALO),
                         lambda bb, g: (bb, 0, jnp.maximum(g * nh - 1, 0))),
            pl.BlockSpec((1, _CIN, _LB), lambda bb, g: (bb, 0, g)),
            pl.BlockSpec((1, _CIN, _HALO),
                         lambda bb, g: (bb, 0,
                                        jnp.minimum((g + 1) * nh, last_halo))),
            pl.BlockSpec((1, 1, _HALO),
                         lambda bb, g: (bb, 0, jnp.maximum(g * nh - 1, 0))),
            pl.BlockSpec((1, 1, _LB), lambda bb, g: (bb, 0, g)),
            pl.BlockSpec((1, 1, _HALO),
                         lambda bb, g: (bb, 0,
                                        jnp.minimum((g + 1) * nh, last_halo))),
            pl.BlockSpec((_COUT, _K * _K * _CIN), lambda bb, g: (0, 0)),
            pl.BlockSpec((_COUT, 1), lambda bb, g: (0, 0)),
        ],
        out_specs=pl.BlockSpec((1, _COUT, _LB), lambda bb, g: (bb, 0, g)),
        out_shape=jax.ShapeDtypeStruct((b, _COUT, _L), jnp.float32),
        scratch_shapes=[pltpu.VMEM((_K * _K * _CIN, _LB), jnp.bfloat16)],
    )(xf, xf, xf, mf, mf, mf, w2, b2)
    return out_flat.reshape(b, _COUT, _H, _W)


# speedup vs baseline: 8.3766x; 1.3948x over previous
import jax
import jax.numpy as jnp
from jax.experimental import pallas as pl
from jax.experimental.pallas import tpu as pltpu

_K = 3
_CIN = 96
_COUT = 96
_H = 224
_W = 224
_R = 32
_LB = _R * _W
_HR = 8
_NG = _H // _R
_NH = _H // _HR - 1


def _conv_body(xp_ref, xm_ref, xn_ref, mp_ref, mm_ref, mn_ref, w_ref, b_ref,
               o_ref, xu_ref):
    g = pl.program_id(1)
    xwin = jnp.concatenate(
        [xp_ref[0, :, _HR - 2:].astype(jnp.bfloat16),
         xm_ref[0].astype(jnp.bfloat16),
         xn_ref[0, :, :2].astype(jnp.bfloat16)], axis=1)
    xflat = xwin.reshape(_CIN, (_R + 4) * _W)
    mwin3 = jnp.concatenate(
        [mp_ref[0, :, _HR - 2:], mm_ref[0], mn_ref[0, :, :2]], axis=1)
    mflat = mwin3.reshape(1, (_R + 4) * _W)

    lane = jax.lax.broadcasted_iota(jnp.int32, (1, _LB), 1)
    col = lane % _W
    j_first = col == 0
    j_last = col == _W - 1
    row_top = jnp.logical_and(g == 0, lane < _W)
    row_bot = jnp.logical_and(g == _NG - 1, lane >= _LB - _W)

    mwin = jnp.zeros((1, _LB), dtype=jnp.float32)
    for di in range(_K):
        for dj in range(_K):
            t = di * _K + dj
            o = (di + 1) * _W + dj - 1
            zm = jnp.zeros((1, _LB), dtype=jnp.bool_)
            if dj == 0:
                zm = j_first
            elif dj == 2:
                zm = j_last
            if di == 0:
                zm = jnp.logical_or(zm, row_top)
            elif di == 2:
                zm = jnp.logical_or(zm, row_bot)
            xs = jnp.where(zm, jnp.bfloat16(0), xflat[:, o:o + _LB])
            ms = jnp.where(zm, 0.0, mflat[:, o:o + _LB])
            xu_ref[t * _CIN:(t + 1) * _CIN, :] = xs
            mwin = jnp.maximum(mwin, jnp.abs(ms))

    acc = jnp.dot(w_ref[...], xu_ref[...], preferred_element_type=jnp.float32)
    out = acc + b_ref[...]
    out = jnp.where(mwin != 0, out, 0.0)
    o_ref[0] = out.reshape(_COUT, _R, _W)


def kernel(x, mask, weight, bias):
    b = x.shape[0]
    w2 = jnp.transpose(weight, (0, 2, 3, 1)).reshape(_COUT, _K * _K * _CIN)
    w2 = w2.astype(jnp.bfloat16)
    b2 = bias.reshape(_COUT, 1)

    nh = _R // _HR

    grid = (b, _NG)
    out = pl.pallas_call(
        _conv_body,
        grid=grid,
        in_specs=[
            pl.BlockSpec((1, _CIN, _HR, _W),
                         lambda bb, g: (bb, 0, jnp.maximum(g * nh - 1, 0), 0)),
            pl.BlockSpec((1, _CIN, _R, _W), lambda bb, g: (bb, 0, g, 0)),
            pl.BlockSpec((1, _CIN, _HR, _W),
                         lambda bb, g: (bb, 0,
                                        jnp.minimum((g + 1) * nh, _NH), 0)),
            pl.BlockSpec((1, 1, _HR, _W),
                         lambda bb, g: (bb, 0, jnp.maximum(g * nh - 1, 0), 0)),
            pl.BlockSpec((1, 1, _R, _W), lambda bb, g: (bb, 0, g, 0)),
            pl.BlockSpec((1, 1, _HR, _W),
                         lambda bb, g: (bb, 0,
                                        jnp.minimum((g + 1) * nh, _NH), 0)),
            pl.BlockSpec((_COUT, _K * _K * _CIN), lambda bb, g: (0, 0)),
            pl.BlockSpec((_COUT, 1), lambda bb, g: (0, 0)),
        ],
        out_specs=pl.BlockSpec((1, _COUT, _R, _W), lambda bb, g: (bb, 0, g, 0)),
        out_shape=jax.ShapeDtypeStruct((b, _COUT, _H, _W), jnp.float32),
        scratch_shapes=[pltpu.VMEM((_K * _K * _CIN, _LB), jnp.bfloat16)],
    )(x, x, x, mask, mask, mask, w2, b2)
    return out


# speedup vs baseline: 8.8539x; 1.0570x over previous
import jax
import jax.numpy as jnp
from jax.experimental import pallas as pl
from jax.experimental.pallas import tpu as pltpu

_K = 3
_CIN = 96
_COUT = 96
_H = 224
_W = 224
_R = 32
_LB = _R * _W
_HR = 8
_NG = _H // _R
_NH = _H // _HR - 1
_KG = _K * _CIN


def _conv_body(xp_ref, xm_ref, xn_ref, mp_ref, mm_ref, mn_ref, w_ref, b_ref,
               o_ref, xua_ref, xub_ref, xuc_ref):
    g = pl.program_id(1)
    xwin = jnp.concatenate(
        [xp_ref[0, :, _HR - 2:].astype(jnp.bfloat16),
         xm_ref[0].astype(jnp.bfloat16),
         xn_ref[0, :, :2].astype(jnp.bfloat16)], axis=1)
    xflat = xwin.reshape(_CIN, (_R + 4) * _W)

    lane = jax.lax.broadcasted_iota(jnp.int32, (1, _LB), 1)
    col = lane % _W
    j_first = col == 0
    j_last = col == _W - 1
    row_top = jnp.logical_and(g == 0, lane < _W)
    row_bot = jnp.logical_and(g == _NG - 1, lane >= _LB - _W)

    groups = (xua_ref, xub_ref, xuc_ref)
    acc = jnp.zeros((_COUT, _LB), dtype=jnp.float32)
    for di in range(_K):
        xu_ref = groups[di]
        for dj in range(_K):
            o = (di + 1) * _W + dj - 1
            zm = None
            if dj == 0:
                zm = j_first
            elif dj == 2:
                zm = j_last
            if di == 0:
                zm = row_top if zm is None else jnp.logical_or(zm, row_top)
            elif di == 2:
                zm = row_bot if zm is None else jnp.logical_or(zm, row_bot)
            xs = xflat[:, o:o + _LB]
            if zm is not None:
                xs = jnp.where(zm, jnp.bfloat16(0), xs)
            xu_ref[dj * _CIN:(dj + 1) * _CIN, :] = xs
        acc += jnp.dot(w_ref[:, di * _KG:(di + 1) * _KG], xu_ref[...],
                       preferred_element_type=jnp.float32)

    m3 = jnp.concatenate(
        [mp_ref[0, 0, _HR - 2:], mm_ref[0, 0], mn_ref[0, 0, :2]], axis=0)
    zc = jnp.zeros((_R + 4, 1), dtype=jnp.float32)
    m3p = jnp.concatenate([zc, m3, zc], axis=1)
    rows = jax.lax.broadcasted_iota(jnp.int32, (_R, _W), 0)
    rt2 = jnp.logical_and(g == 0, rows < 1)
    rb2 = jnp.logical_and(g == _NG - 1, rows >= _R - 1)
    mwin = jnp.zeros((_R, _W), dtype=jnp.float32)
    for di in range(_K):
        for dj in range(_K):
            sl = m3p[1 + di:1 + di + _R, dj:dj + _W]
            if di == 0:
                sl = jnp.where(rt2, 0.0, sl)
            elif di == 2:
                sl = jnp.where(rb2, 0.0, sl)
            mwin = jnp.maximum(mwin, jnp.abs(sl))

    out = (acc + b_ref[...]).reshape(_COUT, _R, _W)
    o_ref[0] = jnp.where(mwin[None] != 0, out, 0.0)


def kernel(x, mask, weight, bias):
    b = x.shape[0]
    w2 = jnp.transpose(weight, (0, 2, 3, 1)).reshape(_COUT, _K * _K * _CIN)
    w2 = w2.astype(jnp.bfloat16)
    b2 = bias.reshape(_COUT, 1)

    nh = _R // _HR

    grid = (b, _NG)
    out = pl.pallas_call(
        _conv_body,
        grid=grid,
        in_specs=[
            pl.BlockSpec((1, _CIN, _HR, _W),
                         lambda bb, g: (bb, 0, jnp.maximum(g * nh - 1, 0), 0)),
            pl.BlockSpec((1, _CIN, _R, _W), lambda bb, g: (bb, 0, g, 0)),
            pl.BlockSpec((1, _CIN, _HR, _W),
                         lambda bb, g: (bb, 0,
                                        jnp.minimum((g + 1) * nh, _NH), 0)),
            pl.BlockSpec((1, 1, _HR, _W),
                         lambda bb, g: (bb, 0, jnp.maximum(g * nh - 1, 0), 0)),
            pl.BlockSpec((1, 1, _R, _W), lambda bb, g: (bb, 0, g, 0)),
            pl.BlockSpec((1, 1, _HR, _W),
                         lambda bb, g: (bb, 0,
                                        jnp.minimum((g + 1) * nh, _NH), 0)),
            pl.BlockSpec((_COUT, _K * _K * _CIN), lambda bb, g: (0, 0)),
            pl.BlockSpec((_COUT, 1), lambda bb, g: (0, 0)),
        ],
        out_specs=pl.BlockSpec((1, _COUT, _R, _W), lambda bb, g: (bb, 0, g, 0)),
        out_shape=jax.ShapeDtypeStruct((b, _COUT, _H, _W), jnp.float32),
        scratch_shapes=[pltpu.VMEM((_KG, _LB), jnp.bfloat16),
                        pltpu.VMEM((_KG, _LB), jnp.bfloat16),
                        pltpu.VMEM((_KG, _LB), jnp.bfloat16)],
    )(x, x, x, mask, mask, mask, w2, b2)
    return out
